# Initial kernel scaffold; baseline (speedup 1.0000x reference)
import jax
import jax.numpy as jnp
from jax.experimental import pallas as pl


def kernel(x, edge_index, edge_weight, h0, W):
    raise NotImplementedError("write your pallas kernel here")


# speedup vs baseline: 5.1991x; 5.1991x over previous
import functools

import jax
import jax.numpy as jnp
from jax import lax
from jax.experimental import pallas as pl
from jax.experimental.pallas import tpu as pltpu
from jax.experimental.pallas import tpu_sc as plsc

ALPHA = 0.1
BETA = 0.5

NC = 2
NS = 16
NW = NC * NS
CHUNK = 128


def _prep_body(x_ref, h0_ref, w_ref, sup_ref, dense_ref):
    sup = (1.0 - ALPHA) * x_ref[...] + ALPHA * h0_ref[...]
    sup_ref[...] = sup
    dense_ref[...] = jnp.dot(sup, w_ref[...], preferred_element_type=jnp.float32)


def _comb_body(p_ref, d_ref, o_ref):
    o_ref[...] = (1.0 - BETA) * (p_ref[0] + p_ref[1]) + BETA * d_ref[...]


def _make_sc_agg(n, d, e):
    ew = e // NW
    full = ew // CHUNK
    tail = ew % CHUNK
    rfull = n // CHUNK
    rtail = n % CHUNK
    rchunks = rfull + (1 if rtail else 0)
    kmax = (rchunks + NS - 1) // NS
    assert e % NW == 0 and tail % 8 == 0 and rtail % 8 == 0

    mesh = plsc.VectorSubcoreMesh(core_axis_name="c", subcore_axis_name="s")

    def body(sup_hbm, col_hbm, row_hbm, w_hbm, out_hbm,
             idxc_v, idxr_v, w_v, rows_v,
             idxc_t, idxr_t, w_t, rows_t,
             agg_sh, sem):
        c = lax.axis_index("c")
        s = lax.axis_index("s")

        def zfill(r, _):
            for b in range(d // 16):
                rows_v[r, pl.ds(b * 16, 16)] = jnp.zeros((16,), jnp.float32)
            return 0
        lax.fori_loop(0, CHUNK, zfill, 0)
        for k in range(kmax):
            ci = k * NS + s

            @pl.when(ci < rfull)
            def _():
                pltpu.sync_copy(rows_v, agg_sh.at[pl.ds(ci * CHUNK, CHUNK)])

            if rtail:
                @pl.when(ci == rfull)
                def _():
                    pltpu.sync_copy(rows_v.at[pl.ds(0, rtail)],
                                    agg_sh.at[pl.ds(rfull * CHUNK, rtail)])
        plsc.subcore_barrier()

        base_w = (c * NS + s) * ew

        def scale_rows(rv, wv, m):
            def group(g, _):
                w16 = wv[pl.ds(g * 16, 16)]
                for j in range(16):
                    wgt = w16[j]
                    i = g * 16 + j
                    for b in range(d // 16):
                        rv[i, pl.ds(b * 16, 16)] = rv[i, pl.ds(b * 16, 16)] * wgt
                return 0
            lax.fori_loop(0, m // 16, group, 0)

        def chunk(t, _):
            base = base_w + t * CHUNK
            pltpu.sync_copy(col_hbm.at[pl.ds(base, CHUNK)], idxc_v)
            pltpu.sync_copy(row_hbm.at[pl.ds(base, CHUNK)], idxr_v)
            pltpu.sync_copy(w_hbm.at[pl.ds(base, CHUNK)], w_v)
            pltpu.async_copy(sup_hbm.at[idxc_v], rows_v, sem).wait()
            scale_rows(rows_v, w_v, CHUNK)
            pltpu.sync_copy(rows_v, agg_sh.at[idxr_v], add=True)
            return 0
        lax.fori_loop(0, full, chunk, 0)

        if tail:
            base = base_w + full * CHUNK
            pltpu.sync_copy(col_hbm.at[pl.ds(base, tail)], idxc_t)
            pltpu.sync_copy(row_hbm.at[pl.ds(base, tail)], idxr_t)
            pltpu.sync_copy(w_hbm.at[pl.ds(base, tail)], w_t)
            pltpu.async_copy(sup_hbm.at[idxc_t], rows_t, sem).wait()
            scale_rows(rows_t, w_t, tail)
            pltpu.sync_copy(rows_t, agg_sh.at[idxr_t], add=True)

        plsc.subcore_barrier()

        for k in range(kmax):
            ci = k * NS + s

            @pl.when(ci < rfull)
            def _():
                pltpu.sync_copy(agg_sh.at[pl.ds(ci * CHUNK, CHUNK)],
                                out_hbm.at[c, pl.ds(ci * CHUNK, CHUNK)])

            if rtail:
                @pl.when(ci == rfull)
                def _():
                    pltpu.sync_copy(agg_sh.at[pl.ds(rfull * CHUNK, rtail)],
                                    out_hbm.at[c, pl.ds(rfull * CHUNK, rtail)])

    return pl.kernel(
        body,
        out_type=jax.ShapeDtypeStruct((NC, n, d), jnp.float32),
        mesh=mesh,
        scratch_types=[
            pltpu.VMEM((CHUNK,), jnp.int32),
            pltpu.VMEM((CHUNK,), jnp.int32),
            pltpu.VMEM((CHUNK,), jnp.float32),
            pltpu.VMEM((CHUNK, d), jnp.float32),
            pltpu.VMEM((max(tail, 8),), jnp.int32),
            pltpu.VMEM((max(tail, 8),), jnp.int32),
            pltpu.VMEM((max(tail, 8),), jnp.float32),
            pltpu.VMEM((max(tail, 8), d), jnp.float32),
            pltpu.VMEM_SHARED((n, d), jnp.float32),
            pltpu.SemaphoreType.DMA,
        ],
    )


def kernel(x, edge_index, edge_weight, h0, W):
    n, d = x.shape
    e = edge_weight.shape[0]
    row = edge_index[0].astype(jnp.int32)
    col = edge_index[1].astype(jnp.int32)

    rb = 1000
    grid = (n // rb,)
    support, dense = pl.pallas_call(
        _prep_body,
        grid=grid,
        in_specs=[
            pl.BlockSpec((rb, d), lambda i: (i, 0)),
            pl.BlockSpec((rb, d), lambda i: (i, 0)),
            pl.BlockSpec((d, d), lambda i: (0, 0)),
        ],
        out_specs=[
            pl.BlockSpec((rb, d), lambda i: (i, 0)),
            pl.BlockSpec((rb, d), lambda i: (i, 0)),
        ],
        out_shape=[
            jax.ShapeDtypeStruct((n, d), jnp.float32),
            jax.ShapeDtypeStruct((n, d), jnp.float32),
        ],
    )(x, h0, W)

    partial = _make_sc_agg(n, d, e)(support, col, row, edge_weight)

    out = pl.pallas_call(
        _comb_body,
        grid=grid,
        in_specs=[
            pl.BlockSpec((NC, rb, d), lambda i: (0, i, 0)),
            pl.BlockSpec((rb, d), lambda i: (i, 0)),
        ],
        out_specs=pl.BlockSpec((rb, d), lambda i: (i, 0)),
        out_shape=jax.ShapeDtypeStruct((n, d), jnp.float32),
    )(partial, dense)
    return out


# speedup vs baseline: 9.5805x; 1.8427x over previous
import functools

import jax
import jax.numpy as jnp
from jax import lax
from jax.experimental import pallas as pl
from jax.experimental.pallas import tpu as pltpu
from jax.experimental.pallas import tpu_sc as plsc

ALPHA = 0.1
BETA = 0.5

NC = 2
NS = 16
NW = NC * NS
CHUNK = 128


def _prep_body(x_ref, h0_ref, w_ref, sup_ref, dense_ref):
    sup = (1.0 - ALPHA) * x_ref[...] + ALPHA * h0_ref[...]
    sup_ref[...] = sup
    dense_ref[...] = jnp.dot(sup, w_ref[...], preferred_element_type=jnp.float32)


def _comb_body(p_ref, d_ref, o_ref):
    o_ref[...] = (1.0 - BETA) * (p_ref[0] + p_ref[1]) + BETA * d_ref[...]


def _make_sc_agg(n, d, e):
    ew = e // NW
    full = ew // CHUNK
    tail = ew % CHUNK
    rfull = n // CHUNK
    rtail = n % CHUNK
    rchunks = rfull + (1 if rtail else 0)
    kmax = (rchunks + NS - 1) // NS
    assert e % NW == 0 and tail % 8 == 0 and rtail % 8 == 0

    mesh = plsc.VectorSubcoreMesh(core_axis_name="c", subcore_axis_name="s")

    assert full % 2 == 0

    def body(sup_hbm, col_hbm, row_hbm, w_hbm, out_hbm,
             idxc_a, idxr_a, w_a, rows_a,
             idxc_b, idxr_b, w_b, rows_b,
             idxc_t, idxr_t, w_t, rows_t,
             agg_sh, sem_ia, sem_ib, sem_ga, sem_gb, sem):
        rows_v = rows_a
        c = lax.axis_index("c")
        s = lax.axis_index("s")

        def zfill(r, _):
            for b in range(d // 16):
                rows_v[r, pl.ds(b * 16, 16)] = jnp.zeros((16,), jnp.float32)
            return 0
        lax.fori_loop(0, CHUNK, zfill, 0)
        for k in range(kmax):
            ci = k * NS + s

            @pl.when(ci < rfull)
            def _():
                pltpu.sync_copy(rows_v, agg_sh.at[pl.ds(ci * CHUNK, CHUNK)])

            if rtail:
                @pl.when(ci == rfull)
                def _():
                    pltpu.sync_copy(rows_v.at[pl.ds(0, rtail)],
                                    agg_sh.at[pl.ds(rfull * CHUNK, rtail)])
        plsc.subcore_barrier()

        base_w = (c * NS + s) * ew

        def scale_rows(rv, wv, m):
            def group(g, _):
                w16 = wv[pl.ds(g * 16, 16)]
                for j in range(16):
                    wgt = w16[j]
                    i = g * 16 + j
                    for b in range(d // 16):
                        rv[i, pl.ds(b * 16, 16)] = rv[i, pl.ds(b * 16, 16)] * wgt
                return 0
            lax.fori_loop(0, m // 16, group, 0)

        def issue_idx(t, ic, ir, iw, s_i):
            base = base_w + t * CHUNK
            pltpu.async_copy(col_hbm.at[pl.ds(base, CHUNK)], ic, s_i)
            pltpu.async_copy(row_hbm.at[pl.ds(base, CHUNK)], ir, s_i)
            pltpu.async_copy(w_hbm.at[pl.ds(base, CHUNK)], iw, s_i)

        def wait_idx(ic, ir, iw, s_i):
            pltpu.make_async_copy(col_hbm.at[pl.ds(0, CHUNK)], ic, s_i).wait()
            pltpu.make_async_copy(row_hbm.at[pl.ds(0, CHUNK)], ir, s_i).wait()
            pltpu.make_async_copy(w_hbm.at[pl.ds(0, CHUNK)], iw, s_i).wait()

        def wait_gather(rv, s_g):
            pltpu.make_async_copy(sup_hbm.at[pl.ds(0, CHUNK)], rv, s_g).wait()

        issue_idx(0, idxc_a, idxr_a, w_a, sem_ia)
        wait_idx(idxc_a, idxr_a, w_a, sem_ia)
        pltpu.async_copy(sup_hbm.at[idxc_a], rows_a, sem_ga)
        issue_idx(1, idxc_b, idxr_b, w_b, sem_ib)

        def chunk_pair(u, _):
            nxa = jnp.minimum(2 * u + 2, full - 1)
            nxb = jnp.minimum(2 * u + 3, full - 1)
            wait_gather(rows_a, sem_ga)
            wait_idx(idxc_b, idxr_b, w_b, sem_ib)
            pltpu.async_copy(sup_hbm.at[idxc_b], rows_b, sem_gb)
            scale_rows(rows_a, w_a, CHUNK)
            pltpu.sync_copy(rows_a, agg_sh.at[idxr_a], add=True)
            issue_idx(nxa, idxc_a, idxr_a, w_a, sem_ia)
            wait_gather(rows_b, sem_gb)
            scale_rows(rows_b, w_b, CHUNK)
            wait_idx(idxc_a, idxr_a, w_a, sem_ia)
            pltpu.async_copy(sup_hbm.at[idxc_a], rows_a, sem_ga)
            pltpu.sync_copy(rows_b, agg_sh.at[idxr_b], add=True)
            issue_idx(nxb, idxc_b, idxr_b, w_b, sem_ib)
            return 0
        lax.fori_loop(0, full // 2, chunk_pair, 0)

        wait_gather(rows_a, sem_ga)
        wait_idx(idxc_b, idxr_b, w_b, sem_ib)

        if tail:
            base = base_w + full * CHUNK
            pltpu.sync_copy(col_hbm.at[pl.ds(base, tail)], idxc_t)
            pltpu.sync_copy(row_hbm.at[pl.ds(base, tail)], idxr_t)
            pltpu.sync_copy(w_hbm.at[pl.ds(base, tail)], w_t)
            pltpu.async_copy(sup_hbm.at[idxc_t], rows_t, sem).wait()
            scale_rows(rows_t, w_t, tail)
            pltpu.sync_copy(rows_t, agg_sh.at[idxr_t], add=True)

        plsc.subcore_barrier()

        for k in range(kmax):
            ci = k * NS + s

            @pl.when(ci < rfull)
            def _():
                pltpu.sync_copy(agg_sh.at[pl.ds(ci * CHUNK, CHUNK)],
                                out_hbm.at[c, pl.ds(ci * CHUNK, CHUNK)])

            if rtail:
                @pl.when(ci == rfull)
                def _():
                    pltpu.sync_copy(agg_sh.at[pl.ds(rfull * CHUNK, rtail)],
                                    out_hbm.at[c, pl.ds(rfull * CHUNK, rtail)])

    return pl.kernel(
        body,
        out_type=jax.ShapeDtypeStruct((NC, n, d), jnp.float32),
        mesh=mesh,
        scratch_types=[
            pltpu.VMEM((CHUNK,), jnp.int32),
            pltpu.VMEM((CHUNK,), jnp.int32),
            pltpu.VMEM((CHUNK,), jnp.float32),
            pltpu.VMEM((CHUNK, d), jnp.float32),
            pltpu.VMEM((CHUNK,), jnp.int32),
            pltpu.VMEM((CHUNK,), jnp.int32),
            pltpu.VMEM((CHUNK,), jnp.float32),
            pltpu.VMEM((CHUNK, d), jnp.float32),
            pltpu.VMEM((max(tail, 8),), jnp.int32),
            pltpu.VMEM((max(tail, 8),), jnp.int32),
            pltpu.VMEM((max(tail, 8),), jnp.float32),
            pltpu.VMEM((max(tail, 8), d), jnp.float32),
            pltpu.VMEM_SHARED((n, d), jnp.float32),
            pltpu.SemaphoreType.DMA,
            pltpu.SemaphoreType.DMA,
            pltpu.SemaphoreType.DMA,
            pltpu.SemaphoreType.DMA,
            pltpu.SemaphoreType.DMA,
        ],
    )


def kernel(x, edge_index, edge_weight, h0, W):
    n, d = x.shape
    e = edge_weight.shape[0]
    row = edge_index[0].astype(jnp.int32)
    col = edge_index[1].astype(jnp.int32)

    rb = 1000
    grid = (n // rb,)
    support, dense = pl.pallas_call(
        _prep_body,
        grid=grid,
        in_specs=[
            pl.BlockSpec((rb, d), lambda i: (i, 0)),
            pl.BlockSpec((rb, d), lambda i: (i, 0)),
            pl.BlockSpec((d, d), lambda i: (0, 0)),
        ],
        out_specs=[
            pl.BlockSpec((rb, d), lambda i: (i, 0)),
            pl.BlockSpec((rb, d), lambda i: (i, 0)),
        ],
        out_shape=[
            jax.ShapeDtypeStruct((n, d), jnp.float32),
            jax.ShapeDtypeStruct((n, d), jnp.float32),
        ],
    )(x, h0, W)

    partial = _make_sc_agg(n, d, e)(support, col, row, edge_weight)

    out = pl.pallas_call(
        _comb_body,
        grid=grid,
        in_specs=[
            pl.BlockSpec((NC, rb, d), lambda i: (0, i, 0)),
            pl.BlockSpec((rb, d), lambda i: (i, 0)),
        ],
        out_specs=pl.BlockSpec((rb, d), lambda i: (i, 0)),
        out_shape=jax.ShapeDtypeStruct((n, d), jnp.float32),
    )(partial, dense)
    return out


# speedup vs baseline: 10.3010x; 1.0752x over previous
import functools

import jax
import jax.numpy as jnp
from jax import lax
from jax.experimental import pallas as pl
from jax.experimental.pallas import tpu as pltpu
from jax.experimental.pallas import tpu_sc as plsc

ALPHA = 0.1
BETA = 0.5

NC = 2
NS = 16
NW = NC * NS
CHUNK = 128


def _prep_body(x_ref, h0_ref, w_ref, sup_ref, dense_ref):
    sup = (1.0 - ALPHA) * x_ref[...] + ALPHA * h0_ref[...]
    sup_ref[...] = sup
    dense_ref[...] = jnp.dot(sup, w_ref[...], preferred_element_type=jnp.float32)


def _comb_body(p_ref, d_ref, o_ref):
    o_ref[...] = (1.0 - BETA) * (p_ref[0] + p_ref[1]) + BETA * d_ref[...]


def _make_sc_agg(n, d, e):
    ew = e // NW
    full = ew // CHUNK
    tail = ew % CHUNK
    rfull = n // CHUNK
    rtail = n % CHUNK
    rchunks = rfull + (1 if rtail else 0)
    kmax = (rchunks + NS - 1) // NS
    assert e % NW == 0 and tail % 8 == 0 and rtail % 8 == 0

    mesh = plsc.VectorSubcoreMesh(core_axis_name="c", subcore_axis_name="s")

    assert full % 2 == 0

    def body(sup_hbm, col_hbm, row_hbm, w_hbm, out_hbm,
             idxc_a, idxr_a, w_a, rows_a,
             idxc_b, idxr_b, w_b, rows_b,
             idxr_sa, idxr_sb,
             idxc_t, idxr_t, w_t, rows_t,
             agg_sh, sem_ia, sem_ib, sem_ga, sem_gb, sem_sa, sem_sb, sem):
        rows_v = rows_a
        c = lax.axis_index("c")
        s = lax.axis_index("s")

        def zfill(r, _):
            for b in range(d // 16):
                rows_v[r, pl.ds(b * 16, 16)] = jnp.zeros((16,), jnp.float32)
            return 0
        lax.fori_loop(0, CHUNK, zfill, 0)
        for k in range(kmax):
            ci = k * NS + s

            @pl.when(ci < rfull)
            def _():
                pltpu.sync_copy(rows_v, agg_sh.at[pl.ds(ci * CHUNK, CHUNK)])

            if rtail:
                @pl.when(ci == rfull)
                def _():
                    pltpu.sync_copy(rows_v.at[pl.ds(0, rtail)],
                                    agg_sh.at[pl.ds(rfull * CHUNK, rtail)])
        plsc.subcore_barrier()

        base_w = (c * NS + s) * ew

        def scale_rows(rv, wv, m):
            def group(g, _):
                w16 = wv[pl.ds(g * 16, 16)]
                for j in range(16):
                    wgt = w16[j]
                    i = g * 16 + j
                    for b in range(d // 16):
                        rv[i, pl.ds(b * 16, 16)] = rv[i, pl.ds(b * 16, 16)] * wgt
                return 0
            lax.fori_loop(0, m // 16, group, 0)

        def issue_idx(t, ic, ir, iw, s_i):
            base = base_w + t * CHUNK
            pltpu.async_copy(col_hbm.at[pl.ds(base, CHUNK)], ic, s_i)
            pltpu.async_copy(row_hbm.at[pl.ds(base, CHUNK)], ir, s_i)
            pltpu.async_copy(w_hbm.at[pl.ds(base, CHUNK)], iw, s_i)

        def wait_idx(ic, ir, iw, s_i):
            pltpu.make_async_copy(col_hbm.at[pl.ds(0, CHUNK)], ic, s_i).wait()
            pltpu.make_async_copy(row_hbm.at[pl.ds(0, CHUNK)], ir, s_i).wait()
            pltpu.make_async_copy(w_hbm.at[pl.ds(0, CHUNK)], iw, s_i).wait()

        def wait_gather(rv, s_g):
            pltpu.make_async_copy(sup_hbm.at[pl.ds(0, CHUNK)], rv, s_g).wait()

        def copy_vec(src, dst):
            for b in range(CHUNK // 16):
                dst[pl.ds(b * 16, 16)] = src[pl.ds(b * 16, 16)]

        def wait_scatter(rv, sidx, s_s):
            pltpu.make_async_copy(rv, agg_sh.at[sidx], s_s).wait()

        def zfill_b(r, _):
            for b in range(d // 16):
                rows_b[r, pl.ds(b * 16, 16)] = jnp.zeros((16,), jnp.float32)
            return 0
        lax.fori_loop(0, CHUNK, zfill_b, 0)
        for b in range(CHUNK // 16):
            idxr_sb[pl.ds(b * 16, 16)] = jnp.zeros((16,), jnp.int32)
        pltpu.async_copy(rows_b, agg_sh.at[idxr_sb], sem_sb, add=True)
        issue_idx(0, idxc_a, idxr_a, w_a, sem_ia)
        wait_idx(idxc_a, idxr_a, w_a, sem_ia)
        pltpu.async_copy(sup_hbm.at[idxc_a], rows_a, sem_ga)
        issue_idx(1, idxc_b, idxr_b, w_b, sem_ib)

        def chunk_pair(u, _):
            nxa = jnp.minimum(2 * u + 2, full - 1)
            nxb = jnp.minimum(2 * u + 3, full - 1)
            wait_scatter(rows_b, idxr_sb, sem_sb)
            wait_idx(idxc_b, idxr_b, w_b, sem_ib)
            pltpu.async_copy(sup_hbm.at[idxc_b], rows_b, sem_gb)
            wait_gather(rows_a, sem_ga)
            scale_rows(rows_a, w_a, CHUNK)
            copy_vec(idxr_a, idxr_sa)
            pltpu.async_copy(rows_a, agg_sh.at[idxr_sa], sem_sa, add=True)
            issue_idx(nxa, idxc_a, idxr_a, w_a, sem_ia)
            wait_gather(rows_b, sem_gb)
            scale_rows(rows_b, w_b, CHUNK)
            wait_scatter(rows_a, idxr_sa, sem_sa)
            wait_idx(idxc_a, idxr_a, w_a, sem_ia)
            pltpu.async_copy(sup_hbm.at[idxc_a], rows_a, sem_ga)
            copy_vec(idxr_b, idxr_sb)
            pltpu.async_copy(rows_b, agg_sh.at[idxr_sb], sem_sb, add=True)
            issue_idx(nxb, idxc_b, idxr_b, w_b, sem_ib)
            return 0
        lax.fori_loop(0, full // 2, chunk_pair, 0)

        wait_scatter(rows_b, idxr_sb, sem_sb)
        wait_gather(rows_a, sem_ga)
        wait_idx(idxc_b, idxr_b, w_b, sem_ib)

        if tail:
            base = base_w + full * CHUNK
            pltpu.sync_copy(col_hbm.at[pl.ds(base, tail)], idxc_t)
            pltpu.sync_copy(row_hbm.at[pl.ds(base, tail)], idxr_t)
            pltpu.sync_copy(w_hbm.at[pl.ds(base, tail)], w_t)
            pltpu.async_copy(sup_hbm.at[idxc_t], rows_t, sem).wait()
            scale_rows(rows_t, w_t, tail)
            pltpu.sync_copy(rows_t, agg_sh.at[idxr_t], add=True)

        plsc.subcore_barrier()

        for k in range(kmax):
            ci = k * NS + s

            @pl.when(ci < rfull)
            def _():
                pltpu.sync_copy(agg_sh.at[pl.ds(ci * CHUNK, CHUNK)],
                                out_hbm.at[c, pl.ds(ci * CHUNK, CHUNK)])

            if rtail:
                @pl.when(ci == rfull)
                def _():
                    pltpu.sync_copy(agg_sh.at[pl.ds(rfull * CHUNK, rtail)],
                                    out_hbm.at[c, pl.ds(rfull * CHUNK, rtail)])

    return pl.kernel(
        body,
        out_type=jax.ShapeDtypeStruct((NC, n, d), jnp.float32),
        mesh=mesh,
        scratch_types=[
            pltpu.VMEM((CHUNK,), jnp.int32),
            pltpu.VMEM((CHUNK,), jnp.int32),
            pltpu.VMEM((CHUNK,), jnp.float32),
            pltpu.VMEM((CHUNK, d), jnp.float32),
            pltpu.VMEM((CHUNK,), jnp.int32),
            pltpu.VMEM((CHUNK,), jnp.int32),
            pltpu.VMEM((CHUNK,), jnp.float32),
            pltpu.VMEM((CHUNK, d), jnp.float32),
            pltpu.VMEM((CHUNK,), jnp.int32),
            pltpu.VMEM((CHUNK,), jnp.int32),
            pltpu.VMEM((max(tail, 8),), jnp.int32),
            pltpu.VMEM((max(tail, 8),), jnp.int32),
            pltpu.VMEM((max(tail, 8),), jnp.float32),
            pltpu.VMEM((max(tail, 8), d), jnp.float32),
            pltpu.VMEM_SHARED((n, d), jnp.float32),
            pltpu.SemaphoreType.DMA,
            pltpu.SemaphoreType.DMA,
            pltpu.SemaphoreType.DMA,
            pltpu.SemaphoreType.DMA,
            pltpu.SemaphoreType.DMA,
            pltpu.SemaphoreType.DMA,
            pltpu.SemaphoreType.DMA,
        ],
    )


def kernel(x, edge_index, edge_weight, h0, W):
    n, d = x.shape
    e = edge_weight.shape[0]
    row = edge_index[0].astype(jnp.int32)
    col = edge_index[1].astype(jnp.int32)

    rb = 1000
    grid = (n // rb,)
    support, dense = pl.pallas_call(
        _prep_body,
        grid=grid,
        in_specs=[
            pl.BlockSpec((rb, d), lambda i: (i, 0)),
            pl.BlockSpec((rb, d), lambda i: (i, 0)),
            pl.BlockSpec((d, d), lambda i: (0, 0)),
        ],
        out_specs=[
            pl.BlockSpec((rb, d), lambda i: (i, 0)),
            pl.BlockSpec((rb, d), lambda i: (i, 0)),
        ],
        out_shape=[
            jax.ShapeDtypeStruct((n, d), jnp.float32),
            jax.ShapeDtypeStruct((n, d), jnp.float32),
        ],
    )(x, h0, W)

    partial = _make_sc_agg(n, d, e)(support, col, row, edge_weight)

    out = pl.pallas_call(
        _comb_body,
        grid=grid,
        in_specs=[
            pl.BlockSpec((NC, rb, d), lambda i: (0, i, 0)),
            pl.BlockSpec((rb, d), lambda i: (i, 0)),
        ],
        out_specs=pl.BlockSpec((rb, d), lambda i: (i, 0)),
        out_shape=jax.ShapeDtypeStruct((n, d), jnp.float32),
    )(partial, dense)
    return out


# speedup vs baseline: 10.6987x; 1.0386x over previous
import jax
import jax.numpy as jnp
from jax import lax
from jax.experimental import pallas as pl
from jax.experimental.pallas import tpu as pltpu
from jax.experimental.pallas import tpu_sc as plsc

ALPHA = 0.1
BETA = 0.5

NC = 2
NS = 16
NW = NC * NS
CHUNK = 128


def _prep_body(x_ref, h0_ref, w_ref, sup_ref, dense_ref):
    sup = (1.0 - ALPHA) * x_ref[...] + ALPHA * h0_ref[...]
    sup_ref[...] = sup
    dense_ref[...] = jnp.dot(sup, w_ref[...], preferred_element_type=jnp.float32)


def _comb_body(p_ref, d_ref, o_ref):
    o_ref[...] = (1.0 - BETA) * (p_ref[0] + p_ref[1]) + BETA * d_ref[...]


def _make_sc_agg(n, d, e):
    nb = e // CHUNK
    nb_w = nb // NW
    nb_r = nb % NW
    rfull = n // CHUNK
    rtail = n % CHUNK
    kmax = (rfull + (1 if rtail else 0) + NS - 1) // NS
    assert e % CHUNK == 0 and rtail % 8 == 0 and nb_w >= 2

    mesh = plsc.VectorSubcoreMesh(core_axis_name="c", subcore_axis_name="s")

    def body(sup_hbm, pk_hbm, out_hbm,
             pk_a, rows_a, pk_b, rows_b, idxr_sa, idxr_sb,
             agg_sh, sem_ia, sem_ib, sem_ga, sem_gb, sem_sa, sem_sb, sem_f):
        c = lax.axis_index("c")
        s = lax.axis_index("s")

        def zfill(r, _):
            for b in range(d // 16):
                rows_a[r, pl.ds(b * 16, 16)] = jnp.zeros((16,), jnp.float32)
            return 0
        lax.fori_loop(0, CHUNK, zfill, 0)
        for k in range(kmax):
            ci = k * NS + s

            @pl.when(ci < rfull)
            def _():
                pltpu.async_copy(rows_a, agg_sh.at[pl.ds(ci * CHUNK, CHUNK)], sem_f)

            if rtail:
                @pl.when(ci == rfull)
                def _():
                    pltpu.async_copy(rows_a.at[pl.ds(0, rtail)],
                                     agg_sh.at[pl.ds(rfull * CHUNK, rtail)], sem_f)
        for k in range(kmax):
            ci = k * NS + s

            @pl.when(ci < rfull)
            def _():
                pltpu.make_async_copy(
                    rows_a, agg_sh.at[pl.ds(0, CHUNK)], sem_f).wait()

            if rtail:
                @pl.when(ci == rfull)
                def _():
                    pltpu.make_async_copy(
                        rows_a.at[pl.ds(0, rtail)],
                        agg_sh.at[pl.ds(0, rtail)], sem_f).wait()
        plsc.subcore_barrier()

        w = c * NS + s
        ntask = nb_w + jnp.where(w < nb_r, 1, 0)
        cstart = w * nb_w + jnp.minimum(w, nb_r)
        clast = cstart + ntask - 1

        def scale_rows(rv, pk):
            def group(g, _):
                w16 = lax.bitcast_convert_type(pk[2, pl.ds(g * 16, 16)],
                                               jnp.float32)
                for j in range(16):
                    wgt = w16[j]
                    i = g * 16 + j
                    for b in range(d // 16):
                        rv[i, pl.ds(b * 16, 16)] = rv[i, pl.ds(b * 16, 16)] * wgt
                return 0
            lax.fori_loop(0, CHUNK // 16, group, 0)

        def copy_ridx(pk, dst):
            for b in range(CHUNK // 16):
                dst[pl.ds(b * 16, 16)] = pk[1, pl.ds(b * 16, 16)]

        def issue_idx(t, pk, s_i):
            pltpu.async_copy(pk_hbm.at[t], pk, s_i)

        def wait_idx(pk, s_i):
            pltpu.make_async_copy(pk_hbm.at[0], pk, s_i).wait()

        def wait_gather(rv, s_g):
            pltpu.make_async_copy(sup_hbm.at[pl.ds(0, CHUNK)], rv, s_g).wait()

        def wait_scatter(rv, sidx, s_s):
            pltpu.make_async_copy(rv, agg_sh.at[sidx], s_s).wait()

        def zfill_b(r, _):
            for b in range(d // 16):
                rows_b[r, pl.ds(b * 16, 16)] = jnp.zeros((16,), jnp.float32)
            return 0
        lax.fori_loop(0, CHUNK, zfill_b, 0)
        for b in range(CHUNK // 16):
            idxr_sb[pl.ds(b * 16, 16)] = jnp.zeros((16,), jnp.int32)
        pltpu.async_copy(rows_b, agg_sh.at[idxr_sb], sem_sb, add=True)
        issue_idx(cstart, pk_a, sem_ia)
        wait_idx(pk_a, sem_ia)
        pltpu.async_copy(sup_hbm.at[pk_a.at[0]], rows_a, sem_ga)
        issue_idx(cstart + 1, pk_b, sem_ib)

        def chunk_pair(u, _):
            nxa = jnp.minimum(cstart + 2 * u + 2, clast)
            nxb = jnp.minimum(cstart + 2 * u + 3, clast)
            wait_scatter(rows_b, idxr_sb, sem_sb)
            wait_idx(pk_b, sem_ib)
            pltpu.async_copy(sup_hbm.at[pk_b.at[0]], rows_b, sem_gb)
            wait_gather(rows_a, sem_ga)
            scale_rows(rows_a, pk_a)
            copy_ridx(pk_a, idxr_sa)
            pltpu.async_copy(rows_a, agg_sh.at[idxr_sa], sem_sa, add=True)
            issue_idx(nxa, pk_a, sem_ia)
            wait_gather(rows_b, sem_gb)
            scale_rows(rows_b, pk_b)
            wait_scatter(rows_a, idxr_sa, sem_sa)
            wait_idx(pk_a, sem_ia)
            pltpu.async_copy(sup_hbm.at[pk_a.at[0]], rows_a, sem_ga)
            copy_ridx(pk_b, idxr_sb)
            pltpu.async_copy(rows_b, agg_sh.at[idxr_sb], sem_sb, add=True)
            issue_idx(nxb, pk_b, sem_ib)
            return 0
        lax.fori_loop(0, ntask // 2, chunk_pair, 0)

        wait_scatter(rows_b, idxr_sb, sem_sb)
        wait_gather(rows_a, sem_ga)
        wait_idx(pk_b, sem_ib)

        @pl.when(ntask % 2 == 1)
        def _():
            scale_rows(rows_a, pk_a)
            copy_ridx(pk_a, idxr_sa)
            pltpu.sync_copy(rows_a, agg_sh.at[idxr_sa], add=True)

        plsc.subcore_barrier()

        for k in range(kmax):
            ci = k * NS + s

            @pl.when(ci < rfull)
            def _():
                pltpu.async_copy(agg_sh.at[pl.ds(ci * CHUNK, CHUNK)],
                                 out_hbm.at[c, pl.ds(ci * CHUNK, CHUNK)], sem_f)

            if rtail:
                @pl.when(ci == rfull)
                def _():
                    pltpu.async_copy(agg_sh.at[pl.ds(rfull * CHUNK, rtail)],
                                     out_hbm.at[c, pl.ds(rfull * CHUNK, rtail)],
                                     sem_f)
        for k in range(kmax):
            ci = k * NS + s

            @pl.when(ci < rfull)
            def _():
                pltpu.make_async_copy(agg_sh.at[pl.ds(0, CHUNK)],
                                      out_hbm.at[0, pl.ds(0, CHUNK)], sem_f).wait()

            if rtail:
                @pl.when(ci == rfull)
                def _():
                    pltpu.make_async_copy(agg_sh.at[pl.ds(0, rtail)],
                                          out_hbm.at[0, pl.ds(0, rtail)],
                                          sem_f).wait()

    return pl.kernel(
        body,
        out_type=jax.ShapeDtypeStruct((NC, n, d), jnp.float32),
        mesh=mesh,
        scratch_types=[
            pltpu.VMEM((3, CHUNK), jnp.int32),
            pltpu.VMEM((CHUNK, d), jnp.float32),
            pltpu.VMEM((3, CHUNK), jnp.int32),
            pltpu.VMEM((CHUNK, d), jnp.float32),
            pltpu.VMEM((CHUNK,), jnp.int32),
            pltpu.VMEM((CHUNK,), jnp.int32),
            pltpu.VMEM_SHARED((n, d), jnp.float32),
            pltpu.SemaphoreType.DMA,
            pltpu.SemaphoreType.DMA,
            pltpu.SemaphoreType.DMA,
            pltpu.SemaphoreType.DMA,
            pltpu.SemaphoreType.DMA,
            pltpu.SemaphoreType.DMA,
            pltpu.SemaphoreType.DMA,
        ],
    )


def kernel(x, edge_index, edge_weight, h0, W):
    n, d = x.shape
    e = edge_weight.shape[0]
    row = edge_index[0].astype(jnp.int32)
    col = edge_index[1].astype(jnp.int32)
    wbits = lax.bitcast_convert_type(edge_weight, jnp.int32)
    packed = (jnp.stack([col, row, wbits], axis=0)
              .reshape(3, e // CHUNK, CHUNK).transpose(1, 0, 2))

    rb = 1000
    grid = (n // rb,)
    support, dense = pl.pallas_call(
        _prep_body,
        grid=grid,
        in_specs=[
            pl.BlockSpec((rb, d), lambda i: (i, 0)),
            pl.BlockSpec((rb, d), lambda i: (i, 0)),
            pl.BlockSpec((d, d), lambda i: (0, 0)),
        ],
        out_specs=[
            pl.BlockSpec((rb, d), lambda i: (i, 0)),
            pl.BlockSpec((rb, d), lambda i: (i, 0)),
        ],
        out_shape=[
            jax.ShapeDtypeStruct((n, d), jnp.float32),
            jax.ShapeDtypeStruct((n, d), jnp.float32),
        ],
    )(x, h0, W)

    partial = _make_sc_agg(n, d, e)(support, packed)

    out = pl.pallas_call(
        _comb_body,
        grid=grid,
        in_specs=[
            pl.BlockSpec((NC, rb, d), lambda i: (0, i, 0)),
            pl.BlockSpec((rb, d), lambda i: (i, 0)),
        ],
        out_specs=pl.BlockSpec((rb, d), lambda i: (i, 0)),
        out_shape=jax.ShapeDtypeStruct((n, d), jnp.float32),
    )(partial, dense)
    return out
